# baseline (speedup 1.0000x reference)
import functools

import jax
import jax.numpy as jnp
from jax import lax
from jax.experimental import pallas as pl
from jax.experimental.pallas import tpu as pltpu
from jax.experimental.pallas import tpu_sc as plsc

L = 16
TILE_R = 8
TILE_C = 128
NC = 2
NS = 16


def _detile_kernel(D, N, NSTEP):
    A = D // TILE_R
    TB = -(-N // TILE_C)
    assert TB % NSTEP == 0
    K = TB // NSTEP
    out_rows = A * TB * TILE_R
    UNROLL = 8

    def body(t_ref, o_ref):
        def move(t):
            src = pl.ds(pl.multiple_of(t * TILE_C, TILE_C), TILE_C)
            dst = pl.ds(pl.multiple_of(t * TILE_R, TILE_R), TILE_R)
            o_ref[dst, :] = t_ref[:, src]

        def run(q, carry):
            for u in range(UNROLL):
                move(q * UNROLL + u)
            return carry

        lax.fori_loop(0, K // UNROLL, run, 0)
        for t in range(K - K % UNROLL, K):
            move(t)

    return pl.pallas_call(
        body,
        grid=(A, NSTEP),
        in_specs=[pl.BlockSpec((TILE_R, TILE_C * K), lambda a, k: (a, k))],
        out_specs=pl.BlockSpec(
            (TILE_R * K, TILE_C), lambda a, k: (a * NSTEP + k, 0)),
        out_shape=jax.ShapeDtypeStruct((out_rows, TILE_C), jnp.float32),
    )


def _sc_mesh():
    return plsc.VectorSubcoreMesh(
        core_axis_name="c", subcore_axis_name="s",
        num_cores=NC, num_subcores=NS)


def _sc_params():
    return pltpu.CompilerParams(
        needs_layout_passes=False, use_tc_tiling_on_sc=False)


def _compute_addresses(ids_ref, w_ref, D, seg, bpw):
    def addr(k, carry):
        sl = pl.ds(k * L, L)
        ids = ids_ref[sl]
        b = ((ids >> 7) << 10) + (ids & (TILE_C - 1))
        for j in range(D):
            cj = (j // TILE_R) * seg + (j % TILE_R) * TILE_C
            w_ref[j, sl] = b + cj
        return carry

    lax.fori_loop(0, bpw // L, addr, 0)


def _gather_user_kernel(B, D, N):
    NW = NC * NS
    bpw = B // NW
    TB = -(-N // TILE_C)
    seg = TB * TILE_R * TILE_C

    @functools.partial(
        pl.kernel,
        out_type=jax.ShapeDtypeStruct((NW, D, bpw), jnp.float32),
        mesh=_sc_mesh(),
        scratch_types=[
            pltpu.VMEM((bpw,), jnp.int32),
            pltpu.VMEM((D, bpw), jnp.int32),
            pltpu.VMEM((D, bpw), jnp.float32),
            pltpu.SemaphoreType.DMA,
        ],
        compiler_params=_sc_params(),
    )
    def gu(uid_hbm, uflat_hbm, out_hbm, uidx_v, uw_v, uf_v, sem):
        wid = lax.axis_index("s") * NC + lax.axis_index("c")
        pltpu.sync_copy(uid_hbm.at[pl.ds(wid * bpw, bpw)], uidx_v)
        _compute_addresses(uidx_v, uw_v, D, seg, bpw)
        copies = [pltpu.async_copy(uflat_hbm.at[uw_v.at[j]], uf_v.at[j], sem)
                  for j in range(D)]
        for c in copies:
            c.wait()
        pltpu.sync_copy(uf_v, out_hbm.at[wid])

    return gu


def _item_dot_kernel(B, D, N):
    NW = NC * NS
    bpw = B // NW
    TB = -(-N // TILE_C)
    seg = TB * TILE_R * TILE_C

    @functools.partial(
        pl.kernel,
        out_type=jax.ShapeDtypeStruct((B,), jnp.float32),
        mesh=_sc_mesh(),
        scratch_types=[
            pltpu.VMEM((bpw,), jnp.int32),
            pltpu.VMEM((D, bpw), jnp.int32),
            pltpu.VMEM((D, bpw), jnp.float32),
            pltpu.VMEM((D, bpw), jnp.float32),
            pltpu.VMEM((bpw,), jnp.float32),
            pltpu.VMEM((bpw,), jnp.float32),
            pltpu.SemaphoreType.DMA,
        ],
        compiler_params=_sc_params(),
    )
    def di(iid_hbm, iflat_hbm, ufeat_hbm, bias_hbm, out_hbm,
           iidx_v, iw_v, if_v, uf_v, bias_v, out_v, sem):
        wid = lax.axis_index("s") * NC + lax.axis_index("c")
        base = wid * bpw
        pltpu.sync_copy(iid_hbm.at[pl.ds(base, bpw)], iidx_v)
        cu = pltpu.async_copy(ufeat_hbm.at[wid], uf_v, sem)
        cb = pltpu.async_copy(bias_hbm.at[iidx_v], bias_v, sem)
        _compute_addresses(iidx_v, iw_v, D, seg, bpw)
        copies = [pltpu.async_copy(iflat_hbm.at[iw_v.at[j]], if_v.at[j], sem)
                  for j in range(D)]
        copies += [cu, cb]
        for c in copies:
            c.wait()

        def group(g, carry):
            sl = pl.ds(g * L, L)
            accs = [jnp.zeros((L,), jnp.float32) for _ in range(4)]
            for j in range(D):
                accs[j % 4] = accs[j % 4] + uf_v[j, sl] * if_v[j, sl]
            out_v[sl] = (accs[0] + accs[1]) + (accs[2] + accs[3]) + bias_v[sl]
            return carry

        lax.fori_loop(0, bpw // L, group, 0)
        pltpu.sync_copy(out_v, out_hbm.at[pl.ds(base, bpw)])

    return di


def kernel(user_ids, item_ids, user_table, item_table, item_bias_table):
    B = user_ids.shape[0]
    N, D = user_table.shape
    bias_flat = item_bias_table.reshape((item_bias_table.shape[0],))
    uid = user_ids.astype(jnp.int32)
    iid = item_ids.astype(jnp.int32)

    detile = _detile_kernel(D, N, 13)

    uimg = detile(user_table.T)
    uflat = uimg.reshape((uimg.shape[0] * uimg.shape[1],))
    ufeat = _gather_user_kernel(B, D, N)(uid, uflat)

    iimg = detile(item_table.T)
    iflat = iimg.reshape((iimg.shape[0] * iimg.shape[1],))
    return _item_dot_kernel(B, D, N)(iid, iflat, ufeat, bias_flat)


# speedup vs baseline: 1.1048x; 1.1048x over previous
import functools

import jax
import jax.numpy as jnp
from jax import lax
from jax.experimental import pallas as pl
from jax.experimental.pallas import tpu as pltpu
from jax.experimental.pallas import tpu_sc as plsc

L = 16
TILE_R = 8
TILE_C = 128


def _detile_kernel(D, N, NSTEP):
    A = D // TILE_R
    TB = -(-N // TILE_C)
    K = -(-TB // NSTEP)
    out_rows = A * NSTEP * K * TILE_R
    UNROLL = 8

    def body(ut_ref, it_ref, uo_ref, io_ref):
        def move(t):
            src = pl.ds(pl.multiple_of(t * TILE_C, TILE_C), TILE_C)
            dst = pl.ds(pl.multiple_of(t * TILE_R, TILE_R), TILE_R)
            uo_ref[dst, :] = ut_ref[:, src]
            io_ref[dst, :] = it_ref[:, src]

        def run(q, carry):
            for u in range(UNROLL):
                move(q * UNROLL + u)
            return carry

        lax.fori_loop(0, K // UNROLL, run, 0)
        for t in range(K - K % UNROLL, K):
            move(t)

    in_spec = pl.BlockSpec((TILE_R, TILE_C * K), lambda a, k: (a, k))
    out_spec = pl.BlockSpec(
        (TILE_R * K, TILE_C), lambda a, k: (a * NSTEP + k, 0))
    return pl.pallas_call(
        body,
        grid=(A, NSTEP),
        in_specs=[in_spec, in_spec],
        out_specs=[out_spec, out_spec],
        out_shape=[
            jax.ShapeDtypeStruct((out_rows, TILE_C), jnp.float32),
            jax.ShapeDtypeStruct((out_rows, TILE_C), jnp.float32),
        ],
    )


def _mf_kernel(B, D, seg, num_cores, num_subcores):
    NW = num_cores * num_subcores
    bpw = B // NW
    mesh = plsc.VectorSubcoreMesh(
        core_axis_name="c", subcore_axis_name="s",
        num_cores=num_cores, num_subcores=num_subcores)

    @functools.partial(
        pl.kernel,
        out_type=jax.ShapeDtypeStruct((B,), jnp.float32),
        mesh=mesh,
        scratch_types=[
            pltpu.VMEM((bpw,), jnp.int32),
            pltpu.VMEM((bpw,), jnp.int32),
            pltpu.VMEM((D, bpw), jnp.int32),
            pltpu.VMEM((D, bpw), jnp.int32),
            pltpu.VMEM((D, bpw), jnp.float32),
            pltpu.VMEM((D, bpw), jnp.float32),
            pltpu.VMEM((bpw,), jnp.float32),
            pltpu.VMEM((bpw,), jnp.float32),
            pltpu.SemaphoreType.DMA,
        ],
        compiler_params=pltpu.CompilerParams(
            needs_layout_passes=False, use_tc_tiling_on_sc=False),
    )
    def mf(uid_hbm, iid_hbm, uflat_hbm, iflat_hbm, bias_hbm, out_hbm,
           uidx_v, iidx_v, uw_v, iw_v, uf_v, if_v, bias_v, out_v, sem):
        wid = lax.axis_index("s") * num_cores + lax.axis_index("c")
        base = wid * bpw

        pltpu.sync_copy(uid_hbm.at[pl.ds(base, bpw)], uidx_v)
        pltpu.sync_copy(iid_hbm.at[pl.ds(base, bpw)], iidx_v)
        cb = pltpu.async_copy(bias_hbm.at[iidx_v], bias_v, sem)

        def addr(k, carry):
            sl = pl.ds(k * L, L)
            for ids_ref, w_ref in ((uidx_v, uw_v), (iidx_v, iw_v)):
                ids = ids_ref[sl]
                b = ((ids >> 7) << 10) + (ids & (TILE_C - 1))
                for j in range(D):
                    cj = (j // TILE_R) * seg + (j % TILE_R) * TILE_C
                    w_ref[j, sl] = b + cj
            return carry

        lax.fori_loop(0, bpw // L, addr, 0)

        copies = [cb]
        for j in range(D):
            copies.append(pltpu.async_copy(
                uflat_hbm.at[uw_v.at[j]], uf_v.at[j], sem))
            copies.append(pltpu.async_copy(
                iflat_hbm.at[iw_v.at[j]], if_v.at[j], sem))
        for c in copies:
            c.wait()

        def group(g, carry):
            sl = pl.ds(g * L, L)
            accs = [jnp.zeros((L,), jnp.float32) for _ in range(4)]
            for j in range(D):
                accs[j % 4] = accs[j % 4] + uf_v[j, sl] * if_v[j, sl]
            out_v[sl] = (accs[0] + accs[1]) + (accs[2] + accs[3]) + bias_v[sl]
            return carry

        lax.fori_loop(0, bpw // L, group, 0)

        pltpu.sync_copy(out_v, out_hbm.at[pl.ds(base, bpw)])

    return mf


def kernel(user_ids, item_ids, user_table, item_table, item_bias_table):
    B = user_ids.shape[0]
    N, D = user_table.shape
    bias_flat = item_bias_table.reshape((item_bias_table.shape[0],))

    uimg, iimg = _detile_kernel(D, N, 8)(user_table.T, item_table.T)
    uflat = uimg.reshape((uimg.shape[0] * uimg.shape[1],))
    iflat = iimg.reshape((iimg.shape[0] * iimg.shape[1],))
    seg = (uimg.shape[0] // (D // TILE_R)) * TILE_C

    mf = _mf_kernel(B, D, seg, 2, 16)
    return mf(user_ids.astype(jnp.int32), item_ids.astype(jnp.int32),
              uflat, iflat, bias_flat)


# speedup vs baseline: 1.1071x; 1.0021x over previous
import functools

import jax
import jax.numpy as jnp
from jax import lax
from jax.experimental import pallas as pl
from jax.experimental.pallas import tpu as pltpu
from jax.experimental.pallas import tpu_sc as plsc

L = 16
TILE_R = 8
TILE_C = 128


def _detile_kernel(D, N, NSTEP):
    A = D // TILE_R
    TB = -(-N // TILE_C)
    K = -(-TB // NSTEP)
    out_rows = A * NSTEP * K * TILE_R
    UNROLL = 8

    def body(ut_ref, it_ref, uo_ref, io_ref):
        for t_ref, o_ref in ((ut_ref, uo_ref), (it_ref, io_ref)):
            x = t_ref[...].reshape(TILE_R, K, TILE_C)
            o_ref[...] = jnp.swapaxes(x, 0, 1).reshape(K * TILE_R, TILE_C)

    in_spec = pl.BlockSpec((TILE_R, TILE_C * K), lambda a, k: (a, k))
    out_spec = pl.BlockSpec(
        (TILE_R * K, TILE_C), lambda a, k: (a * NSTEP + k, 0))
    return pl.pallas_call(
        body,
        grid=(A, NSTEP),
        in_specs=[in_spec, in_spec],
        out_specs=[out_spec, out_spec],
        out_shape=[
            jax.ShapeDtypeStruct((out_rows, TILE_C), jnp.float32),
            jax.ShapeDtypeStruct((out_rows, TILE_C), jnp.float32),
        ],
    )


def _mf_kernel(B, D, seg, num_cores, num_subcores):
    NW = num_cores * num_subcores
    bpw = B // NW
    mesh = plsc.VectorSubcoreMesh(
        core_axis_name="c", subcore_axis_name="s",
        num_cores=num_cores, num_subcores=num_subcores)

    @functools.partial(
        pl.kernel,
        out_type=jax.ShapeDtypeStruct((B,), jnp.float32),
        mesh=mesh,
        scratch_types=[
            pltpu.VMEM((bpw,), jnp.int32),
            pltpu.VMEM((bpw,), jnp.int32),
            pltpu.VMEM((D, bpw), jnp.int32),
            pltpu.VMEM((D, bpw), jnp.int32),
            pltpu.VMEM((D, bpw), jnp.float32),
            pltpu.VMEM((D, bpw), jnp.float32),
            pltpu.VMEM((bpw,), jnp.float32),
            pltpu.VMEM((bpw,), jnp.float32),
            pltpu.SemaphoreType.DMA,
        ],
        compiler_params=pltpu.CompilerParams(
            needs_layout_passes=False, use_tc_tiling_on_sc=False),
    )
    def mf(uid_hbm, iid_hbm, uflat_hbm, iflat_hbm, bias_hbm, out_hbm,
           uidx_v, iidx_v, uw_v, iw_v, uf_v, if_v, bias_v, out_v, sem):
        wid = lax.axis_index("s") * num_cores + lax.axis_index("c")
        base = wid * bpw

        pltpu.sync_copy(uid_hbm.at[pl.ds(base, bpw)], uidx_v)
        pltpu.sync_copy(iid_hbm.at[pl.ds(base, bpw)], iidx_v)
        cb = pltpu.async_copy(bias_hbm.at[iidx_v], bias_v, sem)

        def addr(k, carry):
            sl = pl.ds(k * L, L)
            for ids_ref, w_ref in ((uidx_v, uw_v), (iidx_v, iw_v)):
                ids = ids_ref[sl]
                b = ((ids >> 7) << 10) + (ids & (TILE_C - 1))
                for j in range(D):
                    cj = (j // TILE_R) * seg + (j % TILE_R) * TILE_C
                    w_ref[j, sl] = b + cj
            return carry

        lax.fori_loop(0, bpw // L, addr, 0)

        copies = [cb]
        for j in range(D):
            copies.append(pltpu.async_copy(
                uflat_hbm.at[uw_v.at[j]], uf_v.at[j], sem))
            copies.append(pltpu.async_copy(
                iflat_hbm.at[iw_v.at[j]], if_v.at[j], sem))
        for c in copies:
            c.wait()

        def group(g, carry):
            sl = pl.ds(g * L, L)
            accs = [jnp.zeros((L,), jnp.float32) for _ in range(4)]
            for j in range(D):
                accs[j % 4] = accs[j % 4] + uf_v[j, sl] * if_v[j, sl]
            out_v[sl] = (accs[0] + accs[1]) + (accs[2] + accs[3]) + bias_v[sl]
            return carry

        lax.fori_loop(0, bpw // L, group, 0)

        pltpu.sync_copy(out_v, out_hbm.at[pl.ds(base, bpw)])

    return mf


def kernel(user_ids, item_ids, user_table, item_table, item_bias_table):
    B = user_ids.shape[0]
    N, D = user_table.shape
    bias_flat = item_bias_table.reshape((item_bias_table.shape[0],))

    uimg, iimg = _detile_kernel(D, N, 8)(user_table.T, item_table.T)
    uflat = uimg.reshape((uimg.shape[0] * uimg.shape[1],))
    iflat = iimg.reshape((iimg.shape[0] * iimg.shape[1],))
    seg = (uimg.shape[0] // (D // TILE_R)) * TILE_C

    mf = _mf_kernel(B, D, seg, 2, 16)
    return mf(user_ids.astype(jnp.int32), item_ids.astype(jnp.int32),
              uflat, iflat, bias_flat)
